# baseline (speedup 1.0000x reference)
import functools

import jax
import jax.numpy as jnp
from jax import lax
from jax.experimental import pallas as pl
from jax.experimental.pallas import tpu as pltpu
from jax.experimental.pallas import tpu_sc as plsc

VOCAB_SIZE = 8192
N_ROWS = 8192
NUM_CORES = 2
NUM_SUBCORES = 16
NUM_WORKERS = NUM_CORES * NUM_SUBCORES
ROWS_PER_WORKER = N_ROWS // NUM_WORKERS
LANES = 16
GROUPS = ROWS_PER_WORKER // LANES

SUB = 128
BLOCK_ROWS = 512


def _sc_payload_body(ids_hbm, subrows_hbm, subpos_hbm, ids_v, buf, spbuf, lock):
    del lock
    wid = lax.axis_index("s") * NUM_CORES + lax.axis_index("c")
    base_row = wid * ROWS_PER_WORKER

    pltpu.sync_copy(ids_hbm.at[pl.ds(base_row, ROWS_PER_WORKER)], ids_v)

    lane = lax.broadcasted_iota(jnp.int32, (LANES,), 0)

    for g in range(GROUPS):
        vec = ids_v[pl.ds(g * LANES, LANES)]
        col = vec & (VOCAB_SIZE - 1)
        spbuf[pl.ds(g * LANES, LANES)] = col >> 7

    def row_body(k, carry):
        g16 = (k >> 4) << 4
        vec = ids_v[pl.ds(g16, LANES)]
        inner = vec & (SUB - 1)
        bcast = inner.at[jnp.full((LANES,), k & (LANES - 1), jnp.int32)].get(
            mode="promise_in_bounds"
        )
        for u in range(SUB // LANES):
            val = jnp.where(lane + u * LANES == bcast, 5.0, 0.0).astype(jnp.float32)
            buf[k, pl.ds(u * LANES, LANES)] = val
        return carry

    lax.fori_loop(0, ROWS_PER_WORKER, row_body, 0)

    pltpu.sync_copy(buf, subrows_hbm.at[pl.ds(base_row, ROWS_PER_WORKER)])
    pltpu.sync_copy(spbuf, subpos_hbm.at[pl.ds(base_row, ROWS_PER_WORKER)])


@functools.cache
def _build_sc_payload():
    return pl.kernel(
        _sc_payload_body,
        out_type=(
            jax.ShapeDtypeStruct((N_ROWS, SUB), jnp.float32),
            jax.ShapeDtypeStruct((N_ROWS,), jnp.int32),
        ),
        mesh=plsc.VectorSubcoreMesh(core_axis_name="c", subcore_axis_name="s"),
        scratch_types=[
            pltpu.VMEM((ROWS_PER_WORKER,), jnp.int32),
            pltpu.VMEM((ROWS_PER_WORKER, SUB), jnp.float32),
            pltpu.VMEM((ROWS_PER_WORKER,), jnp.int32),
            pltpu.SemaphoreType.DMA,
        ],
    )


def _tc_assemble_body(subpos_ref, subrows_ref, out_ref):
    out_ref[...] = jnp.zeros((BLOCK_ROWS, VOCAB_SIZE), jnp.float32)
    for r in range(BLOCK_ROWS):
        c = subpos_ref[0, 0, r]
        start = pl.multiple_of(c * SUB, SUB)
        out_ref[r, pl.ds(start, SUB)] = subrows_ref[r, :]


def kernel(input_ids):
    bs, seq = input_ids.shape
    subrows, subpos = _build_sc_payload()(input_ids.reshape(-1))
    out = pl.pallas_call(
        _tc_assemble_body,
        grid=(N_ROWS // BLOCK_ROWS,),
        in_specs=[
            pl.BlockSpec(
                (1, 1, BLOCK_ROWS),
                lambda i: (i, 0, 0),
                memory_space=pltpu.SMEM,
            ),
            pl.BlockSpec((BLOCK_ROWS, SUB), lambda i: (i, 0)),
        ],
        out_specs=pl.BlockSpec((BLOCK_ROWS, VOCAB_SIZE), lambda i: (i, 0)),
        out_shape=jax.ShapeDtypeStruct((N_ROWS, VOCAB_SIZE), jnp.float32),
    )(subpos.reshape(N_ROWS // BLOCK_ROWS, 1, BLOCK_ROWS), subrows)
    return out.reshape(bs, seq, VOCAB_SIZE)


# speedup vs baseline: 1.0221x; 1.0221x over previous
import functools

import jax
import jax.numpy as jnp
from jax import lax
from jax.experimental import pallas as pl
from jax.experimental.pallas import tpu as pltpu
from jax.experimental.pallas import tpu_sc as plsc

VOCAB_SIZE = 8192
N_ROWS = 8192
NUM_CORES = 2
NUM_SUBCORES = 16
NUM_WORKERS = NUM_CORES * NUM_SUBCORES
ROWS_PER_WORKER = N_ROWS // NUM_WORKERS
LANES = 16
GROUPS = ROWS_PER_WORKER // LANES

SUB = 128
BLOCK_ROWS = 256


def _sc_payload_body(ids_hbm, subrows_hbm, subpos_hbm, ids_v, buf, spbuf, lock):
    del lock
    wid = lax.axis_index("s") * NUM_CORES + lax.axis_index("c")
    base_row = wid * ROWS_PER_WORKER

    pltpu.sync_copy(ids_hbm.at[pl.ds(base_row, ROWS_PER_WORKER)], ids_v)

    lane = lax.broadcasted_iota(jnp.int32, (LANES,), 0)

    for g in range(GROUPS):
        vec = ids_v[pl.ds(g * LANES, LANES)]
        col = vec & (VOCAB_SIZE - 1)
        spbuf[pl.ds(g * LANES, LANES)] = col >> 7

    def row_body(k, carry):
        g16 = (k >> 4) << 4
        vec = ids_v[pl.ds(g16, LANES)]
        inner = vec & (SUB - 1)
        bcast = inner.at[jnp.full((LANES,), k & (LANES - 1), jnp.int32)].get(
            mode="promise_in_bounds"
        )
        for u in range(SUB // LANES):
            val = jnp.where(lane + u * LANES == bcast, 5.0, 0.0).astype(jnp.float32)
            buf[k, pl.ds(u * LANES, LANES)] = val
        return carry

    lax.fori_loop(0, ROWS_PER_WORKER, row_body, 0)

    pltpu.sync_copy(buf, subrows_hbm.at[pl.ds(base_row, ROWS_PER_WORKER)])
    pltpu.sync_copy(spbuf, subpos_hbm.at[pl.ds(base_row, ROWS_PER_WORKER)])


@functools.cache
def _build_sc_payload():
    return pl.kernel(
        _sc_payload_body,
        out_type=(
            jax.ShapeDtypeStruct((N_ROWS, SUB), jnp.float32),
            jax.ShapeDtypeStruct((N_ROWS,), jnp.int32),
        ),
        mesh=plsc.VectorSubcoreMesh(core_axis_name="c", subcore_axis_name="s"),
        scratch_types=[
            pltpu.VMEM((ROWS_PER_WORKER,), jnp.int32),
            pltpu.VMEM((ROWS_PER_WORKER, SUB), jnp.float32),
            pltpu.VMEM((ROWS_PER_WORKER,), jnp.int32),
            pltpu.SemaphoreType.DMA,
        ],
    )


def _tc_assemble_body(subpos_ref, subrows_ref, out_ref):
    out_ref[...] = jnp.zeros((BLOCK_ROWS, VOCAB_SIZE), jnp.float32)
    for r in range(BLOCK_ROWS):
        c = subpos_ref[0, 0, r]
        start = pl.multiple_of(c * SUB, SUB)
        out_ref[r, pl.ds(start, SUB)] = subrows_ref[r, :]


def kernel(input_ids):
    bs, seq = input_ids.shape
    subrows, subpos = _build_sc_payload()(input_ids.reshape(-1))
    out = pl.pallas_call(
        _tc_assemble_body,
        grid=(N_ROWS // BLOCK_ROWS,),
        in_specs=[
            pl.BlockSpec(
                (1, 1, BLOCK_ROWS),
                lambda i: (i, 0, 0),
                memory_space=pltpu.SMEM,
            ),
            pl.BlockSpec((BLOCK_ROWS, SUB), lambda i: (i, 0)),
        ],
        out_specs=pl.BlockSpec((BLOCK_ROWS, VOCAB_SIZE), lambda i: (i, 0)),
        out_shape=jax.ShapeDtypeStruct((N_ROWS, VOCAB_SIZE), jnp.float32),
    )(subpos.reshape(N_ROWS // BLOCK_ROWS, 1, BLOCK_ROWS), subrows)
    return out.reshape(bs, seq, VOCAB_SIZE)
